# Initial kernel scaffold; baseline (speedup 1.0000x reference)
import jax
import jax.numpy as jnp
from jax.experimental import pallas as pl


def kernel(z, codebook):
    raise NotImplementedError("write your pallas kernel here")


# speedup vs baseline: 1.1008x; 1.1008x over previous
import functools

import jax
import jax.numpy as jnp
from jax import lax
from jax.experimental import pallas as pl
from jax.experimental.pallas import tpu as pltpu

N_CODES = 8192
CODE_DIM = 64
COMMITMENT_COST = 0.25

_M_BLK = 256


def _argmin_body(z_ref, sz_ref, cb_ref, sc_ref, idx_ref, dsum_ref):
    i = pl.program_id(0)

    z = z_ref[...]
    sz = sz_ref[...]
    mm = lax.dot_general(z, cb_ref[...],
                         (((1,), (1,)), ((), ())),
                         preferred_element_type=jnp.float32)
    d = (sz + sc_ref[...]) - 2.0 * mm

    dmin = jnp.min(d, axis=1, keepdims=True)
    ids = lax.broadcasted_iota(jnp.int32, d.shape, 1)
    idx = jnp.min(jnp.where(d == dmin, ids, jnp.int32(N_CODES)), axis=1)

    idx_ref[...] = idx.reshape(1, 1, _M_BLK)

    @pl.when(i == 0)
    def _():
        dsum_ref[0, 0] = 0.0

    dsum_ref[0, 0] += jnp.sum(dmin)


def _run_argmin(z_flat, sz, codebook, sc_row):
    n_tok = z_flat.shape[0]
    grid = n_tok // _M_BLK
    idx3, dsum = pl.pallas_call(
        _argmin_body,
        grid=(grid,),
        in_specs=[
            pl.BlockSpec((_M_BLK, CODE_DIM), lambda i: (i, 0)),
            pl.BlockSpec((_M_BLK, 1), lambda i: (i, 0)),
            pl.BlockSpec((N_CODES, CODE_DIM), lambda i: (0, 0)),
            pl.BlockSpec((1, N_CODES), lambda i: (0, 0)),
        ],
        out_specs=[
            pl.BlockSpec((1, 1, _M_BLK), lambda i: (i, 0, 0)),
            pl.BlockSpec(memory_space=pltpu.SMEM),
        ],
        out_shape=[
            jax.ShapeDtypeStruct((grid, 1, _M_BLK), jnp.int32),
            jax.ShapeDtypeStruct((1, 1), jnp.float32),
        ],
    )(z_flat, sz, codebook, sc_row)
    return idx3.reshape(n_tok), dsum


def kernel(z, codebook):
    B, N, D = z.shape
    z_flat = z.reshape(-1, D)
    n_tok = z_flat.shape[0]

    sz = jnp.sum(z_flat ** 2, axis=1, keepdims=True)
    sc_row = jnp.sum(codebook ** 2, axis=1).reshape(1, N_CODES)

    indices, dsum = _run_argmin(z_flat, sz, codebook, sc_row)

    z_q_flat = jnp.take(codebook, indices, axis=0)
    z_q = z_q_flat.reshape(B, N, D)
    indices_2d = indices.reshape(B, N)

    mse = dsum[0, 0] / (n_tok * D)
    vq_loss = mse * COMMITMENT_COST + mse

    z_q_st = z + (z_q - z)

    counts = jnp.bincount(indices, length=N_CODES)
    avg_probs = counts.astype(jnp.float32) / n_tok
    perplexity = jnp.exp(-jnp.sum(avg_probs * jnp.log(avg_probs + 1e-10)))
    unique_codes = jnp.sum(counts > 0).astype(jnp.float32)
    usage_ratio = unique_codes / N_CODES

    return (z_q_st, indices_2d, vq_loss, perplexity, usage_ratio, unique_codes)


# speedup vs baseline: 1.2824x; 1.1649x over previous
import functools

import jax
import jax.numpy as jnp
from jax import lax
from jax.experimental import pallas as pl
from jax.experimental.pallas import tpu as pltpu
from jax.experimental.pallas import tpu_sc as plsc

N_CODES = 8192
CODE_DIM = 64
COMMITMENT_COST = 0.25

_M_BLK = 256
_N_TOK = 9216

_NC = 2
_NS = 16
_CHUNK = _N_TOK // (_NC * _NS)
_SUB = 96


def _argmin_body(z2_ref, sz_ref, cb_ref, sc_ref, idx_ref, dsum_ref):
    i = pl.program_id(0)

    mm2 = lax.dot_general(z2_ref[...], cb_ref[...],
                          (((1,), (1,)), ((), ())),
                          preferred_element_type=jnp.float32)
    sz = sz_ref[...]

    m = None
    cidx = None
    for c in range(N_CODES // 128):
        lo, hi = c * 128, (c + 1) * 128
        dch = (sz + sc_ref[:, lo:hi]) - mm2[:, lo:hi]
        if c == 0:
            m = dch
            cidx = jnp.zeros(dch.shape, jnp.int32)
        else:
            upd = dch < m
            cidx = jnp.where(upd, jnp.int32(c), cidx)
            m = jnp.minimum(dch, m)

    dmin = jnp.min(m, axis=1, keepdims=True)
    lane = lax.broadcasted_iota(jnp.int32, m.shape, 1)
    code = cidx * 128 + lane
    sel = jnp.where(m == dmin, code, jnp.int32(N_CODES))
    idx = jnp.min(sel, axis=1)

    idx_ref[...] = idx.reshape(1, 1, _M_BLK)

    @pl.when(i == 0)
    def _():
        dsum_ref[0, 0] = 0.0

    dsum_ref[0, 0] += jnp.sum(dmin)


def _run_argmin(z2_flat, sz, codebook, sc_row):
    grid = _N_TOK // _M_BLK
    idx3, dsum = pl.pallas_call(
        _argmin_body,
        grid=(grid,),
        in_specs=[
            pl.BlockSpec((_M_BLK, CODE_DIM), lambda i: (i, 0)),
            pl.BlockSpec((_M_BLK, 1), lambda i: (i, 0)),
            pl.BlockSpec((N_CODES, CODE_DIM), lambda i: (0, 0)),
            pl.BlockSpec((1, N_CODES), lambda i: (0, 0)),
        ],
        out_specs=[
            pl.BlockSpec((1, 1, _M_BLK), lambda i: (i, 0, 0)),
            pl.BlockSpec(memory_space=pltpu.SMEM),
        ],
        out_shape=[
            jax.ShapeDtypeStruct((grid, 1, _M_BLK), jnp.int32),
            jax.ShapeDtypeStruct((1, 1), jnp.float32),
        ],
    )(z2_flat, sz, codebook, sc_row)
    return idx3.reshape(_N_TOK), dsum


def _sc_body(cb_hbm, idx_hbm, zq_hbm, cnt_hbm,
             idxa, idxb, idxc, rows, ones, zbuf, csh, sem):
    c = lax.axis_index("c")
    s = lax.axis_index("s")
    wid = s * _NC + c
    base = wid * _CHUNK

    zeros16 = jnp.zeros((16,), jnp.float32)
    for i in range(_SC_ZERO // 16):
        zbuf[pl.ds(16 * i, 16)] = zeros16
    ones16 = jnp.full((16,), 1.0, jnp.float32)
    for i in range(_SUB // 16):
        ones[pl.ds(16 * i, 16)] = ones16

    pltpu.sync_copy(zbuf, csh.at[pl.ds(s * _SC_ZERO, _SC_ZERO)])

    pltpu.sync_copy(idx_hbm.at[pl.ds(base, _SUB)], idxa)
    pltpu.sync_copy(idx_hbm.at[pl.ds(base + _SUB, _SUB)], idxb)
    pltpu.sync_copy(idx_hbm.at[pl.ds(base + 2 * _SUB, _SUB)], idxc)

    cp1 = pltpu.async_copy(cb_hbm.at[idxa], rows.at[pl.ds(0, _SUB)], sem)
    cp2 = pltpu.async_copy(cb_hbm.at[idxb], rows.at[pl.ds(_SUB, _SUB)], sem)
    cp3 = pltpu.async_copy(cb_hbm.at[idxc], rows.at[pl.ds(2 * _SUB, _SUB)], sem)
    cp1.wait()
    cp2.wait()
    cp3.wait()

    pltpu.sync_copy(rows, zq_hbm.at[pl.ds(base, _CHUNK)])

    plsc.subcore_barrier()
    pltpu.sync_copy(ones, csh.at[idxa], add=True)
    pltpu.sync_copy(ones, csh.at[idxb], add=True)
    pltpu.sync_copy(ones, csh.at[idxc], add=True)
    plsc.subcore_barrier()

    @pl.when(s == 0)
    def _():
        pltpu.sync_copy(csh, cnt_hbm.at[c])


_SC_ZERO = N_CODES // _NS

_sc_gather_counts = functools.partial(
    pl.kernel,
    out_type=[
        jax.ShapeDtypeStruct((_N_TOK, CODE_DIM), jnp.float32),
        jax.ShapeDtypeStruct((_NC, N_CODES), jnp.float32),
    ],
    mesh=plsc.VectorSubcoreMesh(core_axis_name="c", subcore_axis_name="s"),
    scratch_types=[
        pltpu.VMEM((_SUB,), jnp.int32),
        pltpu.VMEM((_SUB,), jnp.int32),
        pltpu.VMEM((_SUB,), jnp.int32),
        pltpu.VMEM((_CHUNK, CODE_DIM), jnp.float32),
        pltpu.VMEM((_SUB,), jnp.float32),
        pltpu.VMEM((_SC_ZERO,), jnp.float32),
        pltpu.VMEM_SHARED((N_CODES,), jnp.float32),
        pltpu.SemaphoreType.DMA,
    ],
    compiler_params=pltpu.CompilerParams(use_tc_tiling_on_sc=False),
)(_sc_body)


def _stats_body(cnt_ref, dsum_ref, loss_ref, perp_ref, usage_ref, uniq_ref):
    cnt = cnt_ref[...]
    counts = jnp.sum(cnt, axis=0, keepdims=True)
    avg = counts / jnp.float32(_N_TOK)
    ent = avg * jnp.log(avg + 1e-10)
    perp_ref[0, 0] = jnp.exp(-jnp.sum(ent))
    uniq = jnp.sum((counts > 0).astype(jnp.float32))
    uniq_ref[0, 0] = uniq
    usage_ref[0, 0] = uniq / jnp.float32(N_CODES)
    mse = dsum_ref[0, 0] / jnp.float32(_N_TOK * CODE_DIM)
    loss_ref[0, 0] = mse * COMMITMENT_COST + mse


def _run_stats(cnt, dsum):
    return pl.pallas_call(
        _stats_body,
        in_specs=[
            pl.BlockSpec((_NC, N_CODES), lambda: (0, 0)),
            pl.BlockSpec(memory_space=pltpu.SMEM),
        ],
        out_specs=[pl.BlockSpec(memory_space=pltpu.SMEM)] * 4,
        out_shape=[jax.ShapeDtypeStruct((1, 1), jnp.float32)] * 4,
    )(cnt, dsum)


def kernel(z, codebook):
    B, N, D = z.shape
    z_flat = z.reshape(-1, D)

    sz = jnp.sum(z_flat ** 2, axis=1, keepdims=True)
    sc_row = jnp.sum(codebook ** 2, axis=1).reshape(1, N_CODES)
    z2_flat = z_flat * 2.0

    indices, dsum = _run_argmin(z2_flat, sz, codebook, sc_row)

    zq_flat, cnt = _sc_gather_counts(codebook, indices)

    loss, perp, usage, uniq = _run_stats(cnt, dsum)

    z_q = zq_flat.reshape(B, N, D)
    z_q_st = z + (z_q - z)
    indices_2d = indices.reshape(B, N)

    return (z_q_st, indices_2d, loss[0, 0], perp[0, 0],
            usage[0, 0], uniq[0, 0])


# speedup vs baseline: 1.3205x; 1.0297x over previous
import functools

import jax
import jax.numpy as jnp
from jax import lax
from jax.experimental import pallas as pl
from jax.experimental.pallas import tpu as pltpu
from jax.experimental.pallas import tpu_sc as plsc

N_CODES = 8192
CODE_DIM = 64
COMMITMENT_COST = 0.25

_M_BLK = 256
_N_TOK = 9216

_NC = 2
_NS = 16
_CHUNK = _N_TOK // (_NC * _NS)
_SUB = 96


def _argmin_body(z2_ref, sz_ref, cb_ref, sc_ref, idx_ref, dsum_ref):
    i = pl.program_id(0)

    mm2 = lax.dot_general(z2_ref[...], cb_ref[...],
                          (((1,), (1,)), ((), ())),
                          preferred_element_type=jnp.float32)
    sz = sz_ref[...]

    m = None
    cidx = None
    for c in range(N_CODES // 128):
        lo, hi = c * 128, (c + 1) * 128
        dch = (sz + sc_ref[:, lo:hi]) - mm2[:, lo:hi]
        if c == 0:
            m = dch
            cidx = jnp.zeros(dch.shape, jnp.int32)
        else:
            upd = dch < m
            cidx = jnp.where(upd, jnp.int32(c), cidx)
            m = jnp.minimum(dch, m)

    dmin = jnp.min(m, axis=1, keepdims=True)
    lane = lax.broadcasted_iota(jnp.int32, m.shape, 1)
    code = cidx * 128 + lane
    sel = jnp.where(m == dmin, code, jnp.int32(N_CODES))
    idx = jnp.min(sel, axis=1)

    idx_ref[...] = idx.reshape(1, 1, _M_BLK)

    @pl.when(i == 0)
    def _():
        dsum_ref[0, 0] = 0.0

    dsum_ref[0, 0] += jnp.sum(dmin)


def _run_argmin(z2_flat, sz, codebook, sc_row):
    grid = _N_TOK // _M_BLK
    idx3, dsum = pl.pallas_call(
        _argmin_body,
        grid=(grid,),
        in_specs=[
            pl.BlockSpec((_M_BLK, CODE_DIM), lambda i: (i, 0)),
            pl.BlockSpec((_M_BLK, 1), lambda i: (i, 0)),
            pl.BlockSpec((N_CODES, CODE_DIM), lambda i: (0, 0)),
            pl.BlockSpec((1, N_CODES), lambda i: (0, 0)),
        ],
        out_specs=[
            pl.BlockSpec((1, 1, _M_BLK), lambda i: (i, 0, 0)),
            pl.BlockSpec(memory_space=pltpu.SMEM),
        ],
        out_shape=[
            jax.ShapeDtypeStruct((grid, 1, _M_BLK), jnp.int32),
            jax.ShapeDtypeStruct((1, 1), jnp.float32),
        ],
    )(z2_flat, sz, codebook, sc_row)
    return idx3.reshape(_N_TOK), dsum


def _sc_body(cb_hbm, idx_hbm, zq_hbm, cnt_hbm,
             idxa, idxb, idxc, rows, ones, zbuf, csh, sem):
    c = lax.axis_index("c")
    s = lax.axis_index("s")
    wid = s * _NC + c
    base = wid * _CHUNK

    zeros16 = jnp.zeros((16,), jnp.float32)
    for i in range(_SC_ZERO // 16):
        zbuf[pl.ds(16 * i, 16)] = zeros16
    ones16 = jnp.full((16,), 1.0, jnp.float32)
    for i in range(_SUB // 16):
        ones[pl.ds(16 * i, 16)] = ones16

    pltpu.sync_copy(zbuf, csh.at[pl.ds(s * _SC_ZERO, _SC_ZERO)])

    pltpu.sync_copy(idx_hbm.at[pl.ds(base, _SUB)], idxa)
    pltpu.sync_copy(idx_hbm.at[pl.ds(base + _SUB, _SUB)], idxb)
    pltpu.sync_copy(idx_hbm.at[pl.ds(base + 2 * _SUB, _SUB)], idxc)

    cp1 = pltpu.async_copy(cb_hbm.at[idxa], rows.at[pl.ds(0, _SUB)], sem)
    cp2 = pltpu.async_copy(cb_hbm.at[idxb], rows.at[pl.ds(_SUB, _SUB)], sem)
    cp3 = pltpu.async_copy(cb_hbm.at[idxc], rows.at[pl.ds(2 * _SUB, _SUB)], sem)
    cp1.wait()
    cp2.wait()
    cp3.wait()

    pltpu.sync_copy(rows, zq_hbm.at[pl.ds(base, _CHUNK)])

    plsc.subcore_barrier()
    pltpu.sync_copy(ones, csh.at[idxa], add=True)
    pltpu.sync_copy(ones, csh.at[idxb], add=True)
    pltpu.sync_copy(ones, csh.at[idxc], add=True)
    plsc.subcore_barrier()

    @pl.when(s == 0)
    def _():
        pltpu.sync_copy(csh, cnt_hbm.at[c])


_SC_ZERO = N_CODES // _NS

_sc_gather_counts = functools.partial(
    pl.kernel,
    out_type=[
        jax.ShapeDtypeStruct((_N_TOK, CODE_DIM), jnp.float32),
        jax.ShapeDtypeStruct((_NC, N_CODES), jnp.float32),
    ],
    mesh=plsc.VectorSubcoreMesh(core_axis_name="c", subcore_axis_name="s"),
    scratch_types=[
        pltpu.VMEM((_SUB,), jnp.int32),
        pltpu.VMEM((_SUB,), jnp.int32),
        pltpu.VMEM((_SUB,), jnp.int32),
        pltpu.VMEM((_CHUNK, CODE_DIM), jnp.float32),
        pltpu.VMEM((_SUB,), jnp.float32),
        pltpu.VMEM((_SC_ZERO,), jnp.float32),
        pltpu.VMEM_SHARED((N_CODES,), jnp.float32),
        pltpu.SemaphoreType.DMA,
    ],
    compiler_params=pltpu.CompilerParams(use_tc_tiling_on_sc=False),
)(_sc_body)


def _stats_body(cnt_ref, dsum_ref, loss_ref, perp_ref, usage_ref, uniq_ref):
    cnt = cnt_ref[...]
    counts = jnp.sum(cnt, axis=0, keepdims=True)
    avg = counts / jnp.float32(_N_TOK)
    ent = avg * jnp.log(avg + 1e-10)
    perp_ref[0, 0] = jnp.exp(-jnp.sum(ent))
    uniq = jnp.sum((counts > 0).astype(jnp.float32))
    uniq_ref[0, 0] = uniq
    usage_ref[0, 0] = uniq / jnp.float32(N_CODES)
    mse = dsum_ref[0, 0] / jnp.float32(_N_TOK * CODE_DIM)
    loss_ref[0, 0] = mse * COMMITMENT_COST + mse


def _run_stats(cnt, dsum):
    return pl.pallas_call(
        _stats_body,
        in_specs=[
            pl.BlockSpec((_NC, N_CODES), lambda: (0, 0)),
            pl.BlockSpec(memory_space=pltpu.SMEM),
        ],
        out_specs=[pl.BlockSpec(memory_space=pltpu.SMEM)] * 4,
        out_shape=[jax.ShapeDtypeStruct((1, 1), jnp.float32)] * 4,
    )(cnt, dsum)


def kernel(z, codebook):
    B, N, D = z.shape
    z_flat = z.reshape(-1, D)

    sz = jnp.sum(z_flat ** 2, axis=1, keepdims=True)
    sc_row = jnp.sum(codebook ** 2, axis=1).reshape(1, N_CODES)
    z2_flat = z_flat * 2.0

    indices, dsum = _run_argmin(z2_flat, sz, codebook, sc_row)

    zq_flat, cnt = _sc_gather_counts(codebook, indices)

    loss, perp, usage, uniq = _run_stats(cnt, dsum)

    z_q_st = zq_flat.reshape(B, N, D)
    indices_2d = indices.reshape(B, N)

    return (z_q_st, indices_2d, loss[0, 0], perp[0, 0],
            usage[0, 0], uniq[0, 0])


# speedup vs baseline: 1.4037x; 1.0629x over previous
import functools

import jax
import jax.numpy as jnp
from jax import lax
from jax.experimental import pallas as pl
from jax.experimental.pallas import tpu as pltpu
from jax.experimental.pallas import tpu_sc as plsc

N_CODES = 8192
CODE_DIM = 64
COMMITMENT_COST = 0.25

_M_BLK = 512
_N_TOK = 9216

_NC = 2
_NS = 16
_CHUNK = _N_TOK // (_NC * _NS)
_SUB = 96


def _argmin_body(z_ref, sz_ref, cb_ref, sc_ref, idx_ref, dsum_ref):
    i = pl.program_id(0)

    mm2 = lax.dot_general(z_ref[...] * 2.0, cb_ref[...],
                          (((1,), (1,)), ((), ())),
                          preferred_element_type=jnp.float32)
    sz = sz_ref[...]

    m = None
    cidx = None
    for c in range(N_CODES // 128):
        lo, hi = c * 128, (c + 1) * 128
        dch = (sz + sc_ref[:, lo:hi]) - mm2[:, lo:hi]
        if c == 0:
            m = dch
            cidx = jnp.zeros(dch.shape, jnp.int32)
        else:
            upd = dch < m
            cidx = jnp.where(upd, jnp.int32(c), cidx)
            m = jnp.minimum(dch, m)

    dmin = jnp.min(m, axis=1, keepdims=True)
    lane = lax.broadcasted_iota(jnp.int32, m.shape, 1)
    code = cidx * 128 + lane
    sel = jnp.where(m == dmin, code, jnp.int32(N_CODES))
    idx = jnp.min(sel, axis=1)

    idx_ref[...] = idx.reshape(1, 1, _M_BLK)

    @pl.when(i == 0)
    def _():
        dsum_ref[0, 0] = 0.0

    dsum_ref[0, 0] += jnp.sum(dmin)


def _run_argmin(z_flat, sz, codebook, sc_row):
    grid = _N_TOK // _M_BLK
    idx3, dsum = pl.pallas_call(
        _argmin_body,
        grid=(grid,),
        in_specs=[
            pl.BlockSpec((_M_BLK, CODE_DIM), lambda i: (i, 0)),
            pl.BlockSpec((_M_BLK, 1), lambda i: (i, 0)),
            pl.BlockSpec((N_CODES, CODE_DIM), lambda i: (0, 0)),
            pl.BlockSpec((1, N_CODES), lambda i: (0, 0)),
        ],
        out_specs=[
            pl.BlockSpec((1, 1, _M_BLK), lambda i: (i, 0, 0)),
            pl.BlockSpec(memory_space=pltpu.SMEM),
        ],
        out_shape=[
            jax.ShapeDtypeStruct((grid, 1, _M_BLK), jnp.int32),
            jax.ShapeDtypeStruct((1, 1), jnp.float32),
        ],
    )(z_flat, sz, codebook, sc_row)
    return idx3.reshape(_N_TOK), dsum


def _sc_body(cb_hbm, idx_hbm, zq_hbm, cnt_hbm,
             idxa, idxb, idxc, rows, ones, zbuf, csh, sem):
    c = lax.axis_index("c")
    s = lax.axis_index("s")
    wid = s * _NC + c
    base = wid * _CHUNK

    zeros16 = jnp.zeros((16,), jnp.float32)
    for i in range(_SC_ZERO // 16):
        zbuf[pl.ds(16 * i, 16)] = zeros16
    ones16 = jnp.full((16,), 1.0, jnp.float32)
    for i in range(_SUB // 16):
        ones[pl.ds(16 * i, 16)] = ones16

    pltpu.sync_copy(zbuf, csh.at[pl.ds(s * _SC_ZERO, _SC_ZERO)])

    pltpu.sync_copy(idx_hbm.at[pl.ds(base, _SUB)], idxa)
    pltpu.sync_copy(idx_hbm.at[pl.ds(base + _SUB, _SUB)], idxb)
    pltpu.sync_copy(idx_hbm.at[pl.ds(base + 2 * _SUB, _SUB)], idxc)

    cp1 = pltpu.async_copy(cb_hbm.at[idxa], rows.at[pl.ds(0, _SUB)], sem)
    cp2 = pltpu.async_copy(cb_hbm.at[idxb], rows.at[pl.ds(_SUB, _SUB)], sem)
    cp3 = pltpu.async_copy(cb_hbm.at[idxc], rows.at[pl.ds(2 * _SUB, _SUB)], sem)
    cp1.wait()
    cp2.wait()
    cp3.wait()

    pltpu.sync_copy(rows, zq_hbm.at[pl.ds(base, _CHUNK)])

    plsc.subcore_barrier()
    pltpu.sync_copy(ones, csh.at[idxa], add=True)
    pltpu.sync_copy(ones, csh.at[idxb], add=True)
    pltpu.sync_copy(ones, csh.at[idxc], add=True)
    plsc.subcore_barrier()

    @pl.when(s == 0)
    def _():
        pltpu.sync_copy(csh, cnt_hbm.at[c])


_SC_ZERO = N_CODES // _NS

_sc_gather_counts = functools.partial(
    pl.kernel,
    out_type=[
        jax.ShapeDtypeStruct((_N_TOK, CODE_DIM), jnp.float32),
        jax.ShapeDtypeStruct((_NC, N_CODES), jnp.float32),
    ],
    mesh=plsc.VectorSubcoreMesh(core_axis_name="c", subcore_axis_name="s"),
    scratch_types=[
        pltpu.VMEM((_SUB,), jnp.int32),
        pltpu.VMEM((_SUB,), jnp.int32),
        pltpu.VMEM((_SUB,), jnp.int32),
        pltpu.VMEM((_CHUNK, CODE_DIM), jnp.float32),
        pltpu.VMEM((_SUB,), jnp.float32),
        pltpu.VMEM((_SC_ZERO,), jnp.float32),
        pltpu.VMEM_SHARED((N_CODES,), jnp.float32),
        pltpu.SemaphoreType.DMA,
    ],
    compiler_params=pltpu.CompilerParams(use_tc_tiling_on_sc=False),
)(_sc_body)


def _stats_body(cnt_ref, dsum_ref, loss_ref, perp_ref, usage_ref, uniq_ref):
    cnt = cnt_ref[...]
    counts = jnp.sum(cnt, axis=0, keepdims=True)
    avg = counts / jnp.float32(_N_TOK)
    ent = avg * jnp.log(avg + 1e-10)
    perp_ref[0, 0] = jnp.exp(-jnp.sum(ent))
    uniq = jnp.sum((counts > 0).astype(jnp.float32))
    uniq_ref[0, 0] = uniq
    usage_ref[0, 0] = uniq / jnp.float32(N_CODES)
    mse = dsum_ref[0, 0] / jnp.float32(_N_TOK * CODE_DIM)
    loss_ref[0, 0] = mse * COMMITMENT_COST + mse


def _run_stats(cnt, dsum):
    return pl.pallas_call(
        _stats_body,
        in_specs=[
            pl.BlockSpec((_NC, N_CODES), lambda: (0, 0)),
            pl.BlockSpec(memory_space=pltpu.SMEM),
        ],
        out_specs=[pl.BlockSpec(memory_space=pltpu.SMEM)] * 4,
        out_shape=[jax.ShapeDtypeStruct((1, 1), jnp.float32)] * 4,
    )(cnt, dsum)


def kernel(z, codebook):
    B, N, D = z.shape
    z_flat = z.reshape(-1, D)

    sz = jnp.sum(z_flat ** 2, axis=1, keepdims=True)
    sc_row = jnp.sum(codebook ** 2, axis=1).reshape(1, N_CODES)

    indices, dsum = _run_argmin(z_flat, sz, codebook, sc_row)

    zq_flat, cnt = _sc_gather_counts(codebook, indices)

    loss, perp, usage, uniq = _run_stats(cnt, dsum)

    z_q_st = zq_flat.reshape(B, N, D)
    indices_2d = indices.reshape(B, N)

    return (z_q_st, indices_2d, loss[0, 0], perp[0, 0],
            usage[0, 0], uniq[0, 0])


# speedup vs baseline: 1.4226x; 1.0135x over previous
import functools

import jax
import jax.numpy as jnp
from jax import lax
from jax.experimental import pallas as pl
from jax.experimental.pallas import tpu as pltpu
from jax.experimental.pallas import tpu_sc as plsc

N_CODES = 8192
CODE_DIM = 64
COMMITMENT_COST = 0.25

_M_BLK = 1024
_N_TOK = 9216

_NC = 2
_NS = 16
_CHUNK = _N_TOK // (_NC * _NS)
_SUB = 96


def _argmin_body(z_ref, sz_ref, cb_ref, sc_ref, idx_ref, dsum_ref):
    i = pl.program_id(0)

    mm2 = lax.dot_general(z_ref[...] * 2.0, cb_ref[...],
                          (((1,), (1,)), ((), ())),
                          preferred_element_type=jnp.float32)
    sz = sz_ref[...]

    m = None
    cidx = None
    for c in range(N_CODES // 128):
        lo, hi = c * 128, (c + 1) * 128
        dch = (sz + sc_ref[:, lo:hi]) - mm2[:, lo:hi]
        if c == 0:
            m = dch
            cidx = jnp.zeros(dch.shape, jnp.int32)
        else:
            upd = dch < m
            cidx = jnp.where(upd, jnp.int32(c), cidx)
            m = jnp.minimum(dch, m)

    dmin = jnp.min(m, axis=1, keepdims=True)
    lane = lax.broadcasted_iota(jnp.int32, m.shape, 1)
    code = cidx * 128 + lane
    sel = jnp.where(m == dmin, code, jnp.int32(N_CODES))
    idx = jnp.min(sel, axis=1)

    idx_ref[...] = idx.reshape(1, 1, _M_BLK)

    @pl.when(i == 0)
    def _():
        dsum_ref[0, 0] = 0.0

    dsum_ref[0, 0] += jnp.sum(dmin)


def _run_argmin(z_flat, sz, codebook, sc_row):
    grid = _N_TOK // _M_BLK
    idx3, dsum = pl.pallas_call(
        _argmin_body,
        grid=(grid,),
        in_specs=[
            pl.BlockSpec((_M_BLK, CODE_DIM), lambda i: (i, 0)),
            pl.BlockSpec((_M_BLK, 1), lambda i: (i, 0)),
            pl.BlockSpec((N_CODES, CODE_DIM), lambda i: (0, 0)),
            pl.BlockSpec((1, N_CODES), lambda i: (0, 0)),
        ],
        out_specs=[
            pl.BlockSpec((1, 1, _M_BLK), lambda i: (i, 0, 0)),
            pl.BlockSpec(memory_space=pltpu.SMEM),
        ],
        out_shape=[
            jax.ShapeDtypeStruct((grid, 1, _M_BLK), jnp.int32),
            jax.ShapeDtypeStruct((1, 1), jnp.float32),
        ],
    )(z_flat, sz, codebook, sc_row)
    return idx3.reshape(_N_TOK), dsum


def _sc_body(cb_hbm, idx_hbm, zq_hbm, cnt_hbm,
             idxa, idxb, idxc, rows, ones, zbuf, csh, sem):
    c = lax.axis_index("c")
    s = lax.axis_index("s")
    wid = s * _NC + c
    base = wid * _CHUNK

    zeros16 = jnp.zeros((16,), jnp.float32)
    for i in range(_SC_ZERO // 16):
        zbuf[pl.ds(16 * i, 16)] = zeros16
    ones16 = jnp.full((16,), 1.0, jnp.float32)
    for i in range(_SUB // 16):
        ones[pl.ds(16 * i, 16)] = ones16

    pltpu.sync_copy(zbuf, csh.at[pl.ds(s * _SC_ZERO, _SC_ZERO)])

    pltpu.sync_copy(idx_hbm.at[pl.ds(base, _SUB)], idxa)
    pltpu.sync_copy(idx_hbm.at[pl.ds(base + _SUB, _SUB)], idxb)
    pltpu.sync_copy(idx_hbm.at[pl.ds(base + 2 * _SUB, _SUB)], idxc)

    cp1 = pltpu.async_copy(cb_hbm.at[idxa], rows.at[pl.ds(0, _SUB)], sem)
    cp2 = pltpu.async_copy(cb_hbm.at[idxb], rows.at[pl.ds(_SUB, _SUB)], sem)
    cp3 = pltpu.async_copy(cb_hbm.at[idxc], rows.at[pl.ds(2 * _SUB, _SUB)], sem)
    cp1.wait()
    cp2.wait()
    cp3.wait()

    pltpu.sync_copy(rows, zq_hbm.at[pl.ds(base, _CHUNK)])

    plsc.subcore_barrier()
    pltpu.sync_copy(ones, csh.at[idxa], add=True)
    pltpu.sync_copy(ones, csh.at[idxb], add=True)
    pltpu.sync_copy(ones, csh.at[idxc], add=True)
    plsc.subcore_barrier()

    @pl.when(s == 0)
    def _():
        pltpu.sync_copy(csh, cnt_hbm.at[c])


_SC_ZERO = N_CODES // _NS

_sc_gather_counts = functools.partial(
    pl.kernel,
    out_type=[
        jax.ShapeDtypeStruct((_N_TOK, CODE_DIM), jnp.float32),
        jax.ShapeDtypeStruct((_NC, N_CODES), jnp.float32),
    ],
    mesh=plsc.VectorSubcoreMesh(core_axis_name="c", subcore_axis_name="s"),
    scratch_types=[
        pltpu.VMEM((_SUB,), jnp.int32),
        pltpu.VMEM((_SUB,), jnp.int32),
        pltpu.VMEM((_SUB,), jnp.int32),
        pltpu.VMEM((_CHUNK, CODE_DIM), jnp.float32),
        pltpu.VMEM((_SUB,), jnp.float32),
        pltpu.VMEM((_SC_ZERO,), jnp.float32),
        pltpu.VMEM_SHARED((N_CODES,), jnp.float32),
        pltpu.SemaphoreType.DMA,
    ],
    compiler_params=pltpu.CompilerParams(use_tc_tiling_on_sc=False),
)(_sc_body)


def _stats_body(cnt_ref, dsum_ref, loss_ref, perp_ref, usage_ref, uniq_ref):
    cnt = cnt_ref[...]
    counts = jnp.sum(cnt, axis=0, keepdims=True)
    avg = counts / jnp.float32(_N_TOK)
    ent = avg * jnp.log(avg + 1e-10)
    perp_ref[0, 0] = jnp.exp(-jnp.sum(ent))
    uniq = jnp.sum((counts > 0).astype(jnp.float32))
    uniq_ref[0, 0] = uniq
    usage_ref[0, 0] = uniq / jnp.float32(N_CODES)
    mse = dsum_ref[0, 0] / jnp.float32(_N_TOK * CODE_DIM)
    loss_ref[0, 0] = mse * COMMITMENT_COST + mse


def _run_stats(cnt, dsum):
    return pl.pallas_call(
        _stats_body,
        in_specs=[
            pl.BlockSpec((_NC, N_CODES), lambda: (0, 0)),
            pl.BlockSpec(memory_space=pltpu.SMEM),
        ],
        out_specs=[pl.BlockSpec(memory_space=pltpu.SMEM)] * 4,
        out_shape=[jax.ShapeDtypeStruct((1, 1), jnp.float32)] * 4,
    )(cnt, dsum)


def kernel(z, codebook):
    B, N, D = z.shape
    z_flat = z.reshape(-1, D)

    sz = jnp.sum(z_flat ** 2, axis=1, keepdims=True)
    sc_row = jnp.sum(codebook ** 2, axis=1).reshape(1, N_CODES)

    indices, dsum = _run_argmin(z_flat, sz, codebook, sc_row)

    zq_flat, cnt = _sc_gather_counts(codebook, indices)

    loss, perp, usage, uniq = _run_stats(cnt, dsum)

    z_q_st = zq_flat.reshape(B, N, D)
    indices_2d = indices.reshape(B, N)

    return (z_q_st, indices_2d, loss[0, 0], perp[0, 0],
            usage[0, 0], uniq[0, 0])
